# baseline (speedup 1.0000x reference)
import functools

import jax
import jax.numpy as jnp
from jax import lax
from jax.experimental import pallas as pl
from jax.experimental.pallas import tpu as pltpu
from jax.experimental.pallas import tpu_sc as plsc

B, K, C, H, W = 8, 8, 3, 384, 384
NWORKERS = 32
ROWS_PER_WORKER = (B * H) // NWORKERS
TH, TW = 8, 128
NTCOL = W // TW
NCHUNKS = (ROWS_PER_WORKER // TH) * NTCOL
LANES = 16
SUBT = TW // LANES
WORKERS_PER_B = H // ROWS_PER_WORKER

f32 = jnp.float32


def _sc_body(scores_hbm, weights_hbm,
             sv0, sv1, wv0, wv1, isem0, isem1, osem0, osem1):
    cid = lax.axis_index("c")
    sid = lax.axis_index("s")
    wid = cid * 16 + sid
    b = wid // WORKERS_PER_B
    h0 = (wid % WORKERS_PER_B) * ROWS_PER_WORKER

    svs, wvs = (sv0, sv1), (wv0, wv1)
    isems, osems = (isem0, isem1), (osem0, osem1)

    ones = jnp.full((LANES,), 1.0, f32)
    zeros = jnp.full((LANES,), 0.0, f32)

    def tile_slice(t):
        h = h0 + (t // NTCOL) * TH
        w = (t % NTCOL) * TW
        return (pl.ds(h, TH), pl.ds(w, TW))

    def issue_in(t):
        hs, ws = tile_slice(t)
        slot = t % 2
        return [pltpu.async_copy(scores_hbm.at[b, :, hs, ws],
                                 svs[slot], isems[slot])]

    def issue_out(t):
        hs, ws = tile_slice(t)
        slot = t % 2
        return [pltpu.async_copy(wvs[slot], weights_hbm.at[b, :, hs, ws],
                                 osems[slot])]

    def compute(slot):
        sv, wv = svs[slot], wvs[slot]

        def row(r, carry0):
            def step(s, carry2):
                sl = pl.ds(s * LANES, LANES)
                m0 = sv[0, r, sl]
                idx = jnp.zeros((LANES,), jnp.int32)
                for k in range(1, K):
                    vk = sv[k, r, sl]
                    gt = vk > m0
                    m0 = jnp.where(gt, vk, m0)
                    idx = jnp.where(gt, jnp.full((LANES,), k, jnp.int32), idx)
                for k in range(K):
                    kvec = jnp.full((LANES,), k, jnp.int32)
                    wv[k, r, sl] = jnp.where(idx == kvec, ones, zeros)
                return carry2

            lax.fori_loop(0, SUBT, step, 0)
            return carry0

        lax.fori_loop(0, TH, row, 0)

    out_descs = [None, None]
    in_descs = issue_in(0)
    for t in range(NCHUNKS):
        slot = t % 2
        next_descs = issue_in(t + 1) if t + 1 < NCHUNKS else None
        for d in in_descs:
            d.wait()
        if out_descs[slot] is not None:
            for d in out_descs[slot]:
                d.wait()
        compute(slot)
        out_descs[slot] = issue_out(t)
        in_descs = next_descs
    for slot in range(2):
        if out_descs[slot] is not None:
            for d in out_descs[slot]:
                d.wait()


def _sc_weights(selection_scores):
    mesh = plsc.VectorSubcoreMesh(core_axis_name="c", subcore_axis_name="s")
    sc = functools.partial(
        pl.kernel,
        mesh=mesh,
        out_type=jax.ShapeDtypeStruct((B, K, H, W), f32),
        compiler_params=pltpu.CompilerParams(use_tc_tiling_on_sc=True),
        scratch_types=[
            pltpu.VMEM((K, TH, TW), f32), pltpu.VMEM((K, TH, TW), f32),
            pltpu.VMEM((K, TH, TW), f32), pltpu.VMEM((K, TH, TW), f32),
            pltpu.SemaphoreType.DMA, pltpu.SemaphoreType.DMA,
            pltpu.SemaphoreType.DMA, pltpu.SemaphoreType.DMA,
        ],
    )(_sc_body)
    return sc(selection_scores)


def _tc_body(cand_ref, scores_ref, mask_ref, partial_ref, final_ref):
    s = scores_ref[0]
    best = jnp.argmax(s, axis=0)
    kidx = jax.lax.broadcasted_iota(jnp.int32, s.shape, 0)
    onehot = (kidx == best[None]).astype(f32)
    cand = cand_ref[0]
    sel = (cand * onehot[:, None]).sum(axis=0)
    m = mask_ref[0, 0]
    final_ref[0] = partial_ref[0] * m[None] + sel * (1.0 - m[None])


def _tc_final(candidate_images, selection_scores, mask, partial_image, nh=192):
    grid = (B, H // nh)
    return pl.pallas_call(
        _tc_body,
        grid=grid,
        in_specs=[
            pl.BlockSpec((1, K, C, nh, W), lambda b, j: (b, 0, 0, j, 0)),
            pl.BlockSpec((1, K, nh, W), lambda b, j: (b, 0, j, 0)),
            pl.BlockSpec((1, 1, nh, W), lambda b, j: (b, 0, j, 0)),
            pl.BlockSpec((1, C, nh, W), lambda b, j: (b, 0, j, 0)),
        ],
        out_specs=pl.BlockSpec((1, C, nh, W), lambda b, j: (b, 0, j, 0)),
        out_shape=jax.ShapeDtypeStruct((B, C, H, W), f32),
        compiler_params=pltpu.CompilerParams(
            dimension_semantics=("parallel", "arbitrary")),
    )(candidate_images, selection_scores, mask, partial_image)


@jax.jit
def _run(candidate_images, selection_scores, mask, partial_image):
    weights = _sc_weights(selection_scores)
    final = _tc_final(candidate_images, selection_scores, mask, partial_image)
    return (final, weights)


def kernel(candidate_images, selection_scores, mask, partial_image):
    return _run(candidate_images, selection_scores, mask, partial_image)


# speedup vs baseline: 1.4091x; 1.4091x over previous
import functools

import jax
import jax.numpy as jnp
from jax.experimental import pallas as pl


def _body(cand_ref, scores_ref, mask_ref, partial_ref, final_ref, weights_ref):
    s = scores_ref[0]
    K = s.shape[0]
    best = jnp.argmax(s, axis=0)
    kidx = jax.lax.broadcasted_iota(jnp.int32, s.shape, 0)
    onehot = (kidx == best[None]).astype(jnp.float32)
    weights_ref[0] = onehot
    cand = cand_ref[0]
    sel = (cand * onehot[:, None]).sum(axis=0)
    m = mask_ref[0, 0]
    final_ref[0] = partial_ref[0] * m[None] + sel * (1.0 - m[None])


@functools.partial(jax.jit, static_argnames=("nh",))
def _run(candidate_images, selection_scores, mask, partial_image, nh=384):
    B, K, C, H, W = candidate_images.shape
    grid = (B, H // nh)
    out_shapes = (
        jax.ShapeDtypeStruct((B, C, H, W), jnp.float32),
        jax.ShapeDtypeStruct((B, K, H, W), jnp.float32),
    )
    return pl.pallas_call(
        _body,
        grid=grid,
        in_specs=[
            pl.BlockSpec((1, K, C, nh, W), lambda b, j: (b, 0, 0, j, 0)),
            pl.BlockSpec((1, K, nh, W), lambda b, j: (b, 0, j, 0)),
            pl.BlockSpec((1, 1, nh, W), lambda b, j: (b, 0, j, 0)),
            pl.BlockSpec((1, C, nh, W), lambda b, j: (b, 0, j, 0)),
        ],
        out_specs=(
            pl.BlockSpec((1, C, nh, W), lambda b, j: (b, 0, j, 0)),
            pl.BlockSpec((1, K, nh, W), lambda b, j: (b, 0, j, 0)),
        ),
        out_shape=out_shapes,
        compiler_params=__import__("jax.experimental.pallas.tpu", fromlist=["x"]).CompilerParams(dimension_semantics=("parallel", "arbitrary")),
    )(candidate_images, selection_scores, mask, partial_image)


def kernel(candidate_images, selection_scores, mask, partial_image):
    return _run(candidate_images, selection_scores, mask, partial_image)


# speedup vs baseline: 1.4175x; 1.0059x over previous
import functools

import jax
import jax.numpy as jnp
from jax.experimental import pallas as pl


def _body(cand_ref, scores_ref, mask_ref, partial_ref, final_ref, weights_ref):
    s = scores_ref[0]
    K = s.shape[0]
    best = jnp.argmax(s, axis=0)
    kidx = jax.lax.broadcasted_iota(jnp.int32, s.shape, 0)
    onehot = (kidx == best[None]).astype(jnp.float32)
    weights_ref[0] = onehot
    cand = cand_ref[0]
    sel = (cand * onehot[:, None]).sum(axis=0)
    m = mask_ref[0, 0]
    final_ref[0] = partial_ref[0] * m[None] + sel * (1.0 - m[None])


@functools.partial(jax.jit, static_argnames=("nh",))
def _run(candidate_images, selection_scores, mask, partial_image, nh=128):
    B, K, C, H, W = candidate_images.shape
    grid = (B, H // nh)
    out_shapes = (
        jax.ShapeDtypeStruct((B, C, H, W), jnp.float32),
        jax.ShapeDtypeStruct((B, K, H, W), jnp.float32),
    )
    return pl.pallas_call(
        _body,
        grid=grid,
        in_specs=[
            pl.BlockSpec((1, K, C, nh, W), lambda b, j: (b, 0, 0, j, 0)),
            pl.BlockSpec((1, K, nh, W), lambda b, j: (b, 0, j, 0)),
            pl.BlockSpec((1, 1, nh, W), lambda b, j: (b, 0, j, 0)),
            pl.BlockSpec((1, C, nh, W), lambda b, j: (b, 0, j, 0)),
        ],
        out_specs=(
            pl.BlockSpec((1, C, nh, W), lambda b, j: (b, 0, j, 0)),
            pl.BlockSpec((1, K, nh, W), lambda b, j: (b, 0, j, 0)),
        ),
        out_shape=out_shapes,
        compiler_params=__import__("jax.experimental.pallas.tpu", fromlist=["x"]).CompilerParams(dimension_semantics=("parallel", "arbitrary")),
    )(candidate_images, selection_scores, mask, partial_image)


def kernel(candidate_images, selection_scores, mask, partial_image):
    return _run(candidate_images, selection_scores, mask, partial_image)


# speedup vs baseline: 1.4305x; 1.0092x over previous
import functools

import jax
import jax.numpy as jnp
from jax.experimental import pallas as pl
from jax.experimental.pallas import tpu as pltpu


def _body(cand_ref, scores_ref, mask_ref, partial_ref, final_ref, weights_ref):
    s = scores_ref[0]
    best = jnp.argmax(s, axis=0)
    kidx = jax.lax.broadcasted_iota(jnp.int32, s.shape, 0)
    onehot = (kidx == best[None]).astype(jnp.float32)
    weights_ref[0] = onehot
    cand = cand_ref[0]
    sel = (cand * onehot[:, None]).sum(axis=0)
    m = mask_ref[0, 0]
    final_ref[0] = partial_ref[0] * m[None] + sel * (1.0 - m[None])


@functools.partial(jax.jit, static_argnames=("nh",))
def _run(candidate_images, selection_scores, mask, partial_image, nh=192):
    B, K, C, H, W = candidate_images.shape
    grid = (B, H // nh)
    out_shapes = (
        jax.ShapeDtypeStruct((B, C, H, W), jnp.float32),
        jax.ShapeDtypeStruct((B, K, H, W), jnp.float32),
    )
    return pl.pallas_call(
        _body,
        grid=grid,
        in_specs=[
            pl.BlockSpec((1, K, C, nh, W), lambda b, j: (b, 0, 0, j, 0)),
            pl.BlockSpec((1, K, nh, W), lambda b, j: (b, 0, j, 0)),
            pl.BlockSpec((1, 1, nh, W), lambda b, j: (b, 0, j, 0)),
            pl.BlockSpec((1, C, nh, W), lambda b, j: (b, 0, j, 0)),
        ],
        out_specs=(
            pl.BlockSpec((1, C, nh, W), lambda b, j: (b, 0, j, 0)),
            pl.BlockSpec((1, K, nh, W), lambda b, j: (b, 0, j, 0)),
        ),
        out_shape=out_shapes,
        compiler_params=pltpu.CompilerParams(
            dimension_semantics=("parallel", "arbitrary")),
    )(candidate_images, selection_scores, mask, partial_image)


def kernel(candidate_images, selection_scores, mask, partial_image):
    return _run(candidate_images, selection_scores, mask, partial_image)
